# Initial kernel scaffold; baseline (speedup 1.0000x reference)
import jax
import jax.numpy as jnp
from jax.experimental import pallas as pl


def kernel(input_ids, embed_weight, proj_weight):
    raise NotImplementedError("write your pallas kernel here")


# speedup vs baseline: 2.7014x; 2.7014x over previous
import functools

import jax
import jax.numpy as jnp
from jax import lax
from jax.experimental import pallas as pl
from jax.experimental.pallas import tpu as pltpu
from jax.experimental.pallas import tpu_sc as plsc

_FACT = 128
_HIDDEN = 2048
_CHUNK = 128


def _sc_gather(table, ids3, n_workers, n_chunks):
    info = plsc.get_sparse_core_info()
    nc = info.num_cores
    b_per_w = n_chunks * _CHUNK
    total = n_workers * b_per_w
    mesh = plsc.VectorSubcoreMesh(core_axis_name="c", subcore_axis_name="s")

    @functools.partial(
        pl.kernel,
        mesh=mesh,
        out_type=jax.ShapeDtypeStruct((total, _FACT), jnp.float32),
        scratch_types=[
            pltpu.VMEM((n_chunks, _CHUNK), jnp.int32),
            pltpu.VMEM((b_per_w, _FACT), jnp.float32),
            pltpu.SemaphoreType.DMA,
        ],
    )
    def gather_kernel(table_hbm, ids_hbm, out_hbm, idx_v, rows_v, sem):
        wid = lax.axis_index("s") * nc + lax.axis_index("c")
        base = wid * b_per_w
        pltpu.sync_copy(ids_hbm.at[wid], idx_v)
        copies = []
        for j in range(n_chunks):
            copies.append(
                pltpu.async_copy(
                    table_hbm.at[idx_v.at[j]],
                    rows_v.at[pl.ds(j * _CHUNK, _CHUNK)],
                    sem,
                )
            )
        for c in copies:
            c.wait()
        pltpu.sync_copy(rows_v, out_hbm.at[pl.ds(base, b_per_w)])

    return gather_kernel(table, ids3)


def _tc_project(x, w, m_blk):
    m = x.shape[0]

    def mm(x_ref, w_ref, o_ref):
        o_ref[...] = lax.dot_general(
            x_ref[...],
            w_ref[...],
            (((1,), (1,)), ((), ())),
            preferred_element_type=jnp.float32,
        )

    return pl.pallas_call(
        mm,
        grid=(m // m_blk,),
        in_specs=[
            pl.BlockSpec((m_blk, _FACT), lambda i: (i, 0)),
            pl.BlockSpec((_HIDDEN, _FACT), lambda i: (0, 0)),
        ],
        out_specs=pl.BlockSpec((m_blk, _HIDDEN), lambda i: (i, 0)),
        out_shape=jax.ShapeDtypeStruct((m, _HIDDEN), jnp.float32),
    )(x, w)


def kernel(input_ids, embed_weight, proj_weight):
    b, s = input_ids.shape
    total = b * s
    n_workers = 32
    n_chunks = total // (n_workers * _CHUNK)
    ids3 = input_ids.reshape(n_workers, n_chunks, _CHUNK).astype(jnp.int32)
    x = _sc_gather(embed_weight, ids3, n_workers, n_chunks)
    y = _tc_project(x, proj_weight, m_blk=1024)
    return y.reshape(b, s, _HIDDEN)


# speedup vs baseline: 2.7018x; 1.0001x over previous
import functools

import jax
import jax.numpy as jnp
from jax import lax
from jax.experimental import pallas as pl
from jax.experimental.pallas import tpu as pltpu
from jax.experimental.pallas import tpu_sc as plsc

_FACT = 128
_HIDDEN = 2048
_CHUNK = 128


def _sc_gather(table, ids3, n_workers, n_chunks):
    info = plsc.get_sparse_core_info()
    nc = info.num_cores
    b_per_w = n_chunks * _CHUNK
    total = n_workers * b_per_w
    mesh = plsc.VectorSubcoreMesh(core_axis_name="c", subcore_axis_name="s")

    @functools.partial(
        pl.kernel,
        mesh=mesh,
        out_type=jax.ShapeDtypeStruct((total, _FACT), jnp.float32),
        scratch_types=[
            pltpu.VMEM((n_chunks, _CHUNK), jnp.int32),
            pltpu.VMEM((b_per_w, _FACT), jnp.float32),
            pltpu.SemaphoreType.DMA,
        ],
    )
    def gather_kernel(table_hbm, ids_hbm, out_hbm, idx_v, rows_v, sem):
        wid = lax.axis_index("s") * nc + lax.axis_index("c")
        base = wid * b_per_w
        pltpu.sync_copy(ids_hbm.at[wid], idx_v)
        copies = []
        for j in range(n_chunks):
            copies.append(
                pltpu.async_copy(
                    table_hbm.at[idx_v.at[j]],
                    rows_v.at[pl.ds(j * _CHUNK, _CHUNK)],
                    sem,
                )
            )
        for c in copies:
            c.wait()
        pltpu.sync_copy(rows_v, out_hbm.at[pl.ds(base, b_per_w)])

    return gather_kernel(table, ids3)


def _tc_project(x, w, m_blk):
    m = x.shape[0]

    def mm(x_ref, w_ref, o_ref):
        o_ref[...] = lax.dot_general(
            x_ref[...].astype(jnp.bfloat16),
            w_ref[...],
            (((1,), (1,)), ((), ())),
            preferred_element_type=jnp.float32,
        )

    return pl.pallas_call(
        mm,
        grid=(m // m_blk,),
        in_specs=[
            pl.BlockSpec((m_blk, _FACT), lambda i: (i, 0)),
            pl.BlockSpec((_HIDDEN, _FACT), lambda i: (0, 0)),
        ],
        out_specs=pl.BlockSpec((m_blk, _HIDDEN), lambda i: (i, 0)),
        out_shape=jax.ShapeDtypeStruct((m, _HIDDEN), jnp.float32),
    )(x, w.astype(jnp.bfloat16))


def kernel(input_ids, embed_weight, proj_weight):
    b, s = input_ids.shape
    total = b * s
    n_workers = 32
    n_chunks = total // (n_workers * _CHUNK)
    ids3 = input_ids.reshape(n_workers, n_chunks, _CHUNK).astype(jnp.int32)
    x = _sc_gather(embed_weight, ids3, n_workers, n_chunks)
    y = _tc_project(x, proj_weight, m_blk=1024)
    return y.reshape(b, s, _HIDDEN)
